# baseline (speedup 1.0000x reference)
import functools

import jax
import jax.numpy as jnp
from jax import lax
from jax.experimental import pallas as pl
from jax.experimental.pallas import tpu as pltpu
from jax.experimental.pallas import tpu_sc as plsc

AXES_LIMIT = 40.0
RESOLUTION = 0.25
WL = int(AXES_LIMIT * 2 / RESOLUTION)

B = 2
C = 128
N = 20000
NPAD = 20480

NSUB = 16
ROUNDS = C // NSUB
CHUNK = 1024
NCHUNK = NPAD // CHUNK
NGROUP = NCHUNK // 2
GI = CHUNK // 16
SENT = 1 << 18
TBLK = 2048


def _prep_kernel(locT_ref, rc_ref):
    x = locT_ref[0, 0]
    y = locT_ref[0, 1]
    m = (x > -1.0) & (x < 1.0) & (y > -1.0) & (y < 1.0)
    xs = jnp.clip(x, -0.999, 0.999) * AXES_LIMIT
    ys = jnp.clip(y, -0.999, 0.999) * AXES_LIMIT
    row = ((AXES_LIMIT - ys) / RESOLUTION).astype(jnp.int32)
    col = ((AXES_LIMIT + xs) / RESOLUTION).astype(jnp.int32)
    rc_ref[0, 0] = jnp.where(m, row * 512 + col, SENT)


_prep = pl.pallas_call(
    _prep_kernel,
    out_shape=jax.ShapeDtypeStruct((B, 1, NPAD), jnp.int32),
    grid=(B,),
    in_specs=[pl.BlockSpec((1, 2, NPAD), lambda b: (b, 0, 0))],
    out_specs=pl.BlockSpec((1, 1, NPAD), lambda b: (b, 0, 0)),
)


def _sc_body(fmap_hbm, rc_hbm, out_hbm, planebuf, idxchunk, outchunk, stage):
    b = lax.axis_index("c")
    t = lax.axis_index("s")

    def round_body(r, _):
        ch = r * NSUB + t
        pltpu.sync_copy(fmap_hbm.at[b, ch], planebuf)

        def chunk_body(k, _):
            g = k >> 1
            gp = (r * NGROUP + g) & 1
            pltpu.sync_copy(rc_hbm.at[b, 0, pl.ds(k * CHUNK, CHUNK)],
                            idxchunk)

            def g_body(i, _):
                rc = idxchunk[pl.ds(i * 16, 16)]
                outchunk[pl.ds(i * 16, 16)] = plsc.load_gather(
                    planebuf, [(rc >> 9) & 511, rc & 511])
                return 0

            lax.fori_loop(0, GI, g_body, 0)
            pltpu.sync_copy(outchunk,
                            stage.at[gp, t, pl.ds((k & 1) * CHUNK, CHUNK)])

            @pl.when((k & 1) == 1)
            def _flush():
                plsc.subcore_barrier()
                fl = g & 7

                @pl.when(t == fl)
                def _():
                    pltpu.sync_copy(
                        stage.at[gp, pl.ds(0, 8), :],
                        out_hbm.at[pl.ds((b * NSUB + 2 * r) * 8, 8),
                                   pl.ds(g * 2 * CHUNK, 2 * CHUNK)])

                @pl.when(t == 8 + fl)
                def _():
                    pltpu.sync_copy(
                        stage.at[gp, pl.ds(8, 8), :],
                        out_hbm.at[pl.ds((b * NSUB + 2 * r + 1) * 8, 8),
                                   pl.ds(g * 2 * CHUNK, 2 * CHUNK)])

            return 0

        lax.fori_loop(0, NCHUNK, chunk_body, 0)
        return 0

    lax.fori_loop(0, ROUNDS, round_body, 0)


@functools.partial(
    pl.kernel,
    out_type=jax.ShapeDtypeStruct((B * C, NPAD), jnp.float32),
    mesh=plsc.VectorSubcoreMesh(core_axis_name="c", subcore_axis_name="s"),
    compiler_params=pltpu.CompilerParams(needs_layout_passes=False),
    scratch_types=[
        pltpu.VMEM((WL, WL), jnp.float32),
        pltpu.VMEM((CHUNK,), jnp.int32),
        pltpu.VMEM((CHUNK,), jnp.float32),
        pltpu.VMEM_SHARED((2, NSUB, 2 * CHUNK), jnp.float32),
    ],
)
def _sc_gather(fmap_hbm, rc_hbm, out_hbm, *scratch):
    _sc_body(fmap_hbm, rc_hbm, out_hbm, *scratch)


def _tr_kernel(x_ref, rc_ref, empty_ref, o_ref):
    m = rc_ref[0] != SENT
    val = jnp.where(m, x_ref[...], empty_ref[...])
    o_ref[0] = val.T


_transpose = pl.pallas_call(
    _tr_kernel,
    out_shape=jax.ShapeDtypeStruct((B, N, C), jnp.float32),
    grid=(B, NPAD // TBLK),
    in_specs=[
        pl.BlockSpec((C, TBLK), lambda b, k: (b, k)),
        pl.BlockSpec((1, 1, TBLK), lambda b, k: (b, 0, k)),
        pl.BlockSpec((C, 1), lambda b, k: (0, 0)),
    ],
    out_specs=pl.BlockSpec((1, TBLK, C), lambda b, k: (b, k, 0)),
)


def kernel(fmap, loc, empty):
    locT = jnp.pad(loc.transpose(0, 2, 1), ((0, 0), (0, 0), (0, NPAD - N)),
                   constant_values=5.0)
    rc = _prep(locT)
    out_t = _sc_gather(fmap, rc)
    return _transpose(out_t, rc, empty.reshape(C, 1))
